# Initial kernel scaffold; baseline (speedup 1.0000x reference)
import jax
import jax.numpy as jnp
from jax.experimental import pallas as pl


def kernel(x, pos_table):
    raise NotImplementedError("write your pallas kernel here")


# speedup vs baseline: 1.7222x; 1.7222x over previous
import jax
import jax.numpy as jnp
from jax.experimental import pallas as pl

N_PIX = 8192
EMB = 1024
B = 4
BLK = 512


def _add_kernel(x_ref, pos_ref, o_ref):
    o_ref[...] = x_ref[...] + pos_ref[...][None, :, :]


def kernel(x, pos_table):
    grid = (N_PIX // BLK,)
    return pl.pallas_call(
        _add_kernel,
        grid=grid,
        in_specs=[
            pl.BlockSpec((B, BLK, EMB), lambda i: (0, i, 0)),
            pl.BlockSpec((BLK, EMB), lambda i: (i, 0)),
        ],
        out_specs=pl.BlockSpec((B, BLK, EMB), lambda i: (0, i, 0)),
        out_shape=jax.ShapeDtypeStruct((B, N_PIX, EMB), jnp.float32),
    )(x, pos_table)
